# baseline (speedup 1.0000x reference)
import functools

import jax
import jax.numpy as jnp
from jax import lax
from jax.experimental import pallas as pl
from jax.experimental.pallas import tpu as pltpu
from jax.experimental.pallas import tpu_sc as plsc

N = 10000
E = 320000
D = 128
NC = 2
NS = 16
NW = NC * NS
NPAD = 10112
RPT = NPAD // NS
C = 128
CH = 80
EPT = C * CH
NB = NPAD // 128

_MESH = dict(core_axis_name="c", subcore_axis_name="s", num_cores=NC,
             num_subcores=NS)

_f32 = jnp.float32
_i32 = jnp.int32


@functools.partial(
    pl.kernel,
    out_type=jax.ShapeDtypeStruct((NC * NPAD,), _f32),
    mesh=plsc.VectorSubcoreMesh(**_MESH),
    scratch_types=[
        pltpu.VMEM_SHARED((NPAD,), _f32),
        pltpu.VMEM((CH, C), _i32),
        pltpu.VMEM((1, C), _i32),
        pltpu.VMEM((RPT,), _f32),
        pltpu.VMEM((C,), _f32),
    ],
)
def _sc_degree(pkq, degp, acc, pkv, didx, stage, ones):
    c = lax.axis_index("c")
    s = lax.axis_index("s")
    w = c * NS + s
    abase = pl.multiple_of(s * RPT, 8)
    obase = pl.multiple_of(c * NPAD + s * RPT, 8)
    z16 = jnp.zeros((16,), _f32)
    o16 = jnp.full((16,), 1.0, _f32)

    def fill_stage(i, _):
        stage[pl.ds(i * 16, 16)] = z16
        return 0

    lax.fori_loop(0, RPT // 16 + 1, fill_stage, 0)

    def fill_ones(i, _):
        ones[pl.ds(i * 16, 16)] = o16
        return 0

    lax.fori_loop(0, C // 16, fill_ones, 0)
    pltpu.sync_copy(stage.at[pl.ds(0, RPT)], acc.at[pl.ds(abase, RPT)])
    plsc.subcore_barrier()
    pltpu.sync_copy(pkq.at[w], pkv)

    def chunk(j, _):
        for k in range(C // 16):
            didx[0, pl.ds(k * 16, 16)] = (
                lax.shift_right_logical(pkv[j, pl.ds(k * 16, 16)], 14))
        pltpu.sync_copy(ones, acc.at[didx.at[0]], add=True)
        return 0

    lax.fori_loop(0, CH, chunk, 0)
    plsc.subcore_barrier()
    pltpu.sync_copy(acc.at[pl.ds(abase, RPT)], stage.at[pl.ds(0, RPT)])
    pltpu.sync_copy(stage.at[pl.ds(0, RPT)], degp.at[pl.ds(obase, RPT)])


@functools.partial(
    pl.kernel,
    out_type=jax.ShapeDtypeStruct((NC, NPAD, D), _f32),
    mesh=plsc.VectorSubcoreMesh(**_MESH),
    scratch_types=[
        pltpu.VMEM_SHARED((NPAD, D), _f32),
        pltpu.VMEM((CH, C), _i32),
        pltpu.VMEM((2, C), _i32),
        pltpu.VMEM((2, C), _i32),
        pltpu.VMEM((2, C, D), _f32),
        pltpu.SemaphoreType.DMA,
        pltpu.SemaphoreType.DMA,
        pltpu.SemaphoreType.DMA,
        pltpu.SemaphoreType.DMA,
    ],
)
def _sc_hop(t_hbm, pkq, u_hbm, acc, pkv, sidx, didx, rbuf,
            gsem0, gsem1, ssem0, ssem1):
    c = lax.axis_index("c")
    s = lax.axis_index("s")
    w = c * NS + s
    z16 = jnp.zeros((16,), _f32)

    def unpack(j, b):
        for k in range(C // 16):
            p = pkv[j, pl.ds(k * 16, 16)]
            sidx[b, pl.ds(k * 16, 16)] = p & 16383
            didx[b, pl.ds(k * 16, 16)] = lax.shift_right_logical(p, 14)

    def zero_rbuf(i, _):
        rbuf[0, i // 8, pl.ds((i % 8) * 16, 16)] = z16
        return 0

    lax.fori_loop(0, C * D // 16, zero_rbuf, 0)
    base = pl.multiple_of(s * RPT, 8)
    for k in range(4):
        pltpu.sync_copy(rbuf.at[0], acc.at[pl.ds(base + k * 128, 128)])
    pltpu.sync_copy(rbuf.at[0, pl.ds(0, RPT - 512)],
                    acc.at[pl.ds(base + 512, RPT - 512)])
    plsc.subcore_barrier()
    pltpu.sync_copy(pkq.at[w], pkv)

    unpack(0, 0)
    pltpu.async_copy(t_hbm.at[sidx.at[0]], rbuf.at[0], gsem0)

    def pair(i, _):
        j0 = 2 * i
        pltpu.make_async_copy(t_hbm.at[sidx.at[0]], rbuf.at[0], gsem0).wait()
        pltpu.async_copy(rbuf.at[0], acc.at[didx.at[0]], ssem0, add=True)

        @pl.when(j0 >= 1)
        def _():
            pltpu.make_async_copy(rbuf.at[1], acc.at[didx.at[1]],
                                  ssem1).wait()

        unpack(j0 + 1, 1)
        pltpu.async_copy(t_hbm.at[sidx.at[1]], rbuf.at[1], gsem1)
        pltpu.make_async_copy(t_hbm.at[sidx.at[1]], rbuf.at[1], gsem1).wait()
        pltpu.async_copy(rbuf.at[1], acc.at[didx.at[1]], ssem1, add=True)
        pltpu.make_async_copy(rbuf.at[0], acc.at[didx.at[0]], ssem0).wait()

        @pl.when(j0 + 2 < CH)
        def _():
            unpack(j0 + 2, 0)
            pltpu.async_copy(t_hbm.at[sidx.at[0]], rbuf.at[0], gsem0)

        return 0

    lax.fori_loop(0, CH // 2, pair, 0)
    pltpu.make_async_copy(rbuf.at[1], acc.at[didx.at[1]], ssem1).wait()
    plsc.subcore_barrier()
    for k in range(4):
        pltpu.sync_copy(acc.at[pl.ds(base + k * 128, 128)], rbuf.at[0])
        pltpu.sync_copy(rbuf.at[0], u_hbm.at[c, pl.ds(base + k * 128, 128)])
    pltpu.sync_copy(acc.at[pl.ds(base + 512, RPT - 512)],
                    rbuf.at[0, pl.ds(0, RPT - 512)])
    pltpu.sync_copy(rbuf.at[0, pl.ds(0, RPT - 512)],
                    u_hbm.at[c, pl.ds(base + 512, RPT - 512)])


def _scale1_body(x_ref, wt_ref, d0_ref, d1_ref, t0_ref, dinv_ref):
    i = pl.program_id(0)
    deg = d0_ref[:] + d1_ref[:] + 1.0
    dinv = lax.rsqrt(deg)
    rows = lax.broadcasted_iota(_i32, (128, 1), 0) + i * 128
    mask = rows < N
    xw = jnp.dot(x_ref[:], wt_ref[:], preferred_element_type=_f32)
    t0_ref[:] = jnp.where(mask, xw * dinv[:, None], 0.0)
    dinv_ref[:] = jnp.where(mask[:, 0], dinv, 0.0)


def _tc_scale1(x, wt, d0, d1):
    return pl.pallas_call(
        _scale1_body,
        grid=(NB,),
        in_specs=[
            pl.BlockSpec((128, D), lambda i: (i, 0)),
            pl.BlockSpec((D, D), lambda i: (0, 0)),
            pl.BlockSpec((128,), lambda i: (i,)),
            pl.BlockSpec((128,), lambda i: (i,)),
        ],
        out_specs=[
            pl.BlockSpec((128, D), lambda i: (i, 0)),
            pl.BlockSpec((128,), lambda i: (i,)),
        ],
        out_shape=[
            jax.ShapeDtypeStruct((NPAD, D), _f32),
            jax.ShapeDtypeStruct((NPAD,), _f32),
        ],
    )(x, wt, d0, d1)


def _scale2_body(dinv_ref, u0_ref, u1_ref, t0_ref, t1_ref):
    dinv = dinv_ref[:]
    s = dinv * dinv
    t1_ref[:] = s[:, None] * (u0_ref[0] + u1_ref[0] + t0_ref[:])


def _tc_scale2(dinv, u, t0):
    return pl.pallas_call(
        _scale2_body,
        grid=(NB,),
        in_specs=[
            pl.BlockSpec((128,), lambda i: (i,)),
            pl.BlockSpec((1, 128, D), lambda i: (0, i, 0)),
            pl.BlockSpec((1, 128, D), lambda i: (1, i, 0)),
            pl.BlockSpec((128, D), lambda i: (i, 0)),
        ],
        out_specs=pl.BlockSpec((128, D), lambda i: (i, 0)),
        out_shape=jax.ShapeDtypeStruct((NPAD, D), _f32),
    )(dinv, u, u, t0)


def _out_body(dinv_ref, q0_ref, q1_ref, t1_ref, b_ref, o_ref):
    dinv = dinv_ref[:]
    o_ref[:] = dinv[:, None] * (q0_ref[0] + q1_ref[0] + t1_ref[:]) + b_ref[:][None, :]


def _tc_out(dinv, q, t1, b):
    return pl.pallas_call(
        _out_body,
        grid=(NB,),
        in_specs=[
            pl.BlockSpec((128,), lambda i: (i,)),
            pl.BlockSpec((1, 128, D), lambda i: (0, i, 0)),
            pl.BlockSpec((1, 128, D), lambda i: (1, i, 0)),
            pl.BlockSpec((128, D), lambda i: (i, 0)),
            pl.BlockSpec((D,), lambda i: (0,)),
        ],
        out_specs=pl.BlockSpec((128, D), lambda i: (i, 0)),
        out_shape=jax.ShapeDtypeStruct((N, D), _f32),
    )(dinv, q, q, t1, b)


def kernel(x, edge_index, W, b):
    src = edge_index[0].astype(_i32)
    dst = edge_index[1].astype(_i32)
    npadrows = NPAD - N
    pad = N + (jnp.arange(NW * EPT - E, dtype=_i32) % npadrows)
    srcp = jnp.concatenate([src, pad])
    dstp = jnp.concatenate([dst, pad])
    pkq = ((dstp << 14) | srcp).reshape(NW, CH, C)
    wt = W.T

    degp = _sc_degree(pkq)
    t0, dinv = _tc_scale1(x, wt, degp[:NPAD], degp[NPAD:])
    u = _sc_hop(t0, pkq)
    t1 = _tc_scale2(dinv, u, t0)
    q = _sc_hop(t1, pkq)
    return _tc_out(dinv, q, t1, b)


# speedup vs baseline: 1.0125x; 1.0125x over previous
import functools

import jax
import jax.numpy as jnp
from jax import lax
from jax.experimental import pallas as pl
from jax.experimental.pallas import tpu as pltpu
from jax.experimental.pallas import tpu_sc as plsc

N = 10000
E = 320000
D = 128
NC = 2
NS = 16
NW = NC * NS
NPAD = 10112
RPT = NPAD // NS
C = 128
CH = 80
EPT = C * CH
NB = NPAD // 128

_MESH = dict(core_axis_name="c", subcore_axis_name="s", num_cores=NC,
             num_subcores=NS)

_f32 = jnp.float32
_i32 = jnp.int32


@functools.partial(
    pl.kernel,
    out_type=jax.ShapeDtypeStruct((NC * NPAD,), _f32),
    mesh=plsc.VectorSubcoreMesh(**_MESH),
    scratch_types=[
        pltpu.VMEM_SHARED((NPAD,), _f32),
        pltpu.VMEM((CH, C), _i32),
        pltpu.VMEM((2, C), _i32),
        pltpu.VMEM((RPT,), _f32),
        pltpu.VMEM((C,), _f32),
        pltpu.SemaphoreType.DMA,
        pltpu.SemaphoreType.DMA,
    ],
)
def _sc_degree(pkq, degp, acc, pkv, didx, stage, ones, ssem0, ssem1):
    c = lax.axis_index("c")
    s = lax.axis_index("s")
    w = c * NS + s
    abase = pl.multiple_of(s * RPT, 8)
    obase = pl.multiple_of(c * NPAD + s * RPT, 8)
    z16 = jnp.zeros((16,), _f32)
    o16 = jnp.full((16,), 1.0, _f32)

    def fill_stage(i, _):
        stage[pl.ds(i * 16, 16)] = z16
        return 0

    lax.fori_loop(0, RPT // 16 + 1, fill_stage, 0)

    def fill_ones(i, _):
        ones[pl.ds(i * 16, 16)] = o16
        return 0

    lax.fori_loop(0, C // 16, fill_ones, 0)
    pltpu.sync_copy(stage.at[pl.ds(0, RPT)], acc.at[pl.ds(abase, RPT)])
    plsc.subcore_barrier()
    pltpu.sync_copy(pkq.at[w], pkv)

    def unpack_dst(j, b):
        for k in range(C // 16):
            didx[b, pl.ds(k * 16, 16)] = (
                lax.shift_right_logical(pkv[j, pl.ds(k * 16, 16)], 14))

    unpack_dst(0, 0)
    pltpu.async_copy(ones, acc.at[didx.at[0]], ssem0, add=True)

    def pair(i, _):
        j0 = 2 * i

        @pl.when(j0 >= 1)
        def _():
            pltpu.make_async_copy(ones, acc.at[didx.at[1]], ssem1).wait()

        unpack_dst(j0 + 1, 1)
        pltpu.async_copy(ones, acc.at[didx.at[1]], ssem1, add=True)
        pltpu.make_async_copy(ones, acc.at[didx.at[0]], ssem0).wait()

        @pl.when(j0 + 2 < CH)
        def _():
            unpack_dst(j0 + 2, 0)
            pltpu.async_copy(ones, acc.at[didx.at[0]], ssem0, add=True)

        return 0

    lax.fori_loop(0, CH // 2, pair, 0)
    pltpu.make_async_copy(ones, acc.at[didx.at[1]], ssem1).wait()
    plsc.subcore_barrier()
    pltpu.sync_copy(acc.at[pl.ds(abase, RPT)], stage.at[pl.ds(0, RPT)])
    pltpu.sync_copy(stage.at[pl.ds(0, RPT)], degp.at[pl.ds(obase, RPT)])


@functools.partial(
    pl.kernel,
    out_type=jax.ShapeDtypeStruct((NC, NPAD, D), _f32),
    mesh=plsc.VectorSubcoreMesh(**_MESH),
    scratch_types=[
        pltpu.VMEM_SHARED((NPAD, D), _f32),
        pltpu.VMEM((CH, C), _i32),
        pltpu.VMEM((2, C), _i32),
        pltpu.VMEM((2, C), _i32),
        pltpu.VMEM((2, C, D), _f32),
        pltpu.SemaphoreType.DMA,
        pltpu.SemaphoreType.DMA,
        pltpu.SemaphoreType.DMA,
        pltpu.SemaphoreType.DMA,
    ],
)
def _sc_hop(t_hbm, pkq, u_hbm, acc, pkv, sidx, didx, rbuf,
            gsem0, gsem1, ssem0, ssem1):
    c = lax.axis_index("c")
    s = lax.axis_index("s")
    w = c * NS + s
    z16 = jnp.zeros((16,), _f32)

    def unpack(j, b):
        for k in range(C // 16):
            p = pkv[j, pl.ds(k * 16, 16)]
            sidx[b, pl.ds(k * 16, 16)] = p & 16383
            didx[b, pl.ds(k * 16, 16)] = lax.shift_right_logical(p, 14)

    def zero_rbuf(i, _):
        rbuf[0, i // 8, pl.ds((i % 8) * 16, 16)] = z16
        return 0

    lax.fori_loop(0, C * D // 16, zero_rbuf, 0)
    base = pl.multiple_of(s * RPT, 8)
    for k in range(4):
        pltpu.sync_copy(rbuf.at[0], acc.at[pl.ds(base + k * 128, 128)])
    pltpu.sync_copy(rbuf.at[0, pl.ds(0, RPT - 512)],
                    acc.at[pl.ds(base + 512, RPT - 512)])
    plsc.subcore_barrier()
    pltpu.sync_copy(pkq.at[w], pkv)

    unpack(0, 0)
    pltpu.async_copy(t_hbm.at[sidx.at[0]], rbuf.at[0], gsem0)

    def pair(i, _):
        j0 = 2 * i
        pltpu.make_async_copy(t_hbm.at[sidx.at[0]], rbuf.at[0], gsem0).wait()
        pltpu.async_copy(rbuf.at[0], acc.at[didx.at[0]], ssem0, add=True)

        @pl.when(j0 >= 1)
        def _():
            pltpu.make_async_copy(rbuf.at[1], acc.at[didx.at[1]],
                                  ssem1).wait()

        unpack(j0 + 1, 1)
        pltpu.async_copy(t_hbm.at[sidx.at[1]], rbuf.at[1], gsem1)
        pltpu.make_async_copy(t_hbm.at[sidx.at[1]], rbuf.at[1], gsem1).wait()
        pltpu.async_copy(rbuf.at[1], acc.at[didx.at[1]], ssem1, add=True)
        pltpu.make_async_copy(rbuf.at[0], acc.at[didx.at[0]], ssem0).wait()

        @pl.when(j0 + 2 < CH)
        def _():
            unpack(j0 + 2, 0)
            pltpu.async_copy(t_hbm.at[sidx.at[0]], rbuf.at[0], gsem0)

        return 0

    lax.fori_loop(0, CH // 2, pair, 0)
    pltpu.make_async_copy(rbuf.at[1], acc.at[didx.at[1]], ssem1).wait()
    plsc.subcore_barrier()
    for k in range(4):
        pltpu.sync_copy(acc.at[pl.ds(base + k * 128, 128)], rbuf.at[0])
        pltpu.sync_copy(rbuf.at[0], u_hbm.at[c, pl.ds(base + k * 128, 128)])
    pltpu.sync_copy(acc.at[pl.ds(base + 512, RPT - 512)],
                    rbuf.at[0, pl.ds(0, RPT - 512)])
    pltpu.sync_copy(rbuf.at[0, pl.ds(0, RPT - 512)],
                    u_hbm.at[c, pl.ds(base + 512, RPT - 512)])


def _scale1_body(x_ref, wt_ref, d0_ref, d1_ref, t0_ref, dinv_ref):
    i = pl.program_id(0)
    deg = d0_ref[:] + d1_ref[:] + 1.0
    dinv = lax.rsqrt(deg)
    rows = lax.broadcasted_iota(_i32, (128, 1), 0) + i * 128
    mask = rows < N
    xw = jnp.dot(x_ref[:], wt_ref[:], preferred_element_type=_f32)
    t0_ref[:] = jnp.where(mask, xw * dinv[:, None], 0.0)
    dinv_ref[:] = jnp.where(mask[:, 0], dinv, 0.0)


def _tc_scale1(x, wt, degp):
    return pl.pallas_call(
        _scale1_body,
        grid=(NB,),
        in_specs=[
            pl.BlockSpec((128, D), lambda i: (i, 0)),
            pl.BlockSpec((D, D), lambda i: (0, 0)),
            pl.BlockSpec((128,), lambda i: (i,)),
            pl.BlockSpec((128,), lambda i: (i + NB,)),
        ],
        out_specs=[
            pl.BlockSpec((128, D), lambda i: (i, 0)),
            pl.BlockSpec((128,), lambda i: (i,)),
        ],
        out_shape=[
            jax.ShapeDtypeStruct((NPAD, D), _f32),
            jax.ShapeDtypeStruct((NPAD,), _f32),
        ],
    )(x, wt, degp, degp)


def _scale2_body(dinv_ref, u0_ref, u1_ref, t0_ref, t1_ref):
    dinv = dinv_ref[:]
    s = dinv * dinv
    t1_ref[:] = s[:, None] * (u0_ref[0] + u1_ref[0] + t0_ref[:])


def _tc_scale2(dinv, u, t0):
    return pl.pallas_call(
        _scale2_body,
        grid=(NB,),
        in_specs=[
            pl.BlockSpec((128,), lambda i: (i,)),
            pl.BlockSpec((1, 128, D), lambda i: (0, i, 0)),
            pl.BlockSpec((1, 128, D), lambda i: (1, i, 0)),
            pl.BlockSpec((128, D), lambda i: (i, 0)),
        ],
        out_specs=pl.BlockSpec((128, D), lambda i: (i, 0)),
        out_shape=jax.ShapeDtypeStruct((NPAD, D), _f32),
    )(dinv, u, u, t0)


def _out_body(dinv_ref, q0_ref, q1_ref, t1_ref, b_ref, o_ref):
    dinv = dinv_ref[:]
    o_ref[:] = dinv[:, None] * (q0_ref[0] + q1_ref[0] + t1_ref[:]) + b_ref[:][None, :]


def _tc_out(dinv, q, t1, b):
    return pl.pallas_call(
        _out_body,
        grid=(NB,),
        in_specs=[
            pl.BlockSpec((128,), lambda i: (i,)),
            pl.BlockSpec((1, 128, D), lambda i: (0, i, 0)),
            pl.BlockSpec((1, 128, D), lambda i: (1, i, 0)),
            pl.BlockSpec((128, D), lambda i: (i, 0)),
            pl.BlockSpec((D,), lambda i: (0,)),
        ],
        out_specs=pl.BlockSpec((128, D), lambda i: (i, 0)),
        out_shape=jax.ShapeDtypeStruct((N, D), _f32),
    )(dinv, q, q, t1, b)


def kernel(x, edge_index, W, b):
    src = edge_index[0].astype(_i32)
    dst = edge_index[1].astype(_i32)
    npadrows = NPAD - N
    pad = N + (jnp.arange(NW * EPT - E, dtype=_i32) % npadrows)
    srcp = jnp.concatenate([src, pad])
    dstp = jnp.concatenate([dst, pad])
    pkq = ((dstp << 14) | srcp).reshape(NW, CH, C)
    wt = W.T

    degp = _sc_degree(pkq)
    t0, dinv = _tc_scale1(x, wt, degp)
    u = _sc_hop(t0, pkq)
    t1 = _tc_scale2(dinv, u, t0)
    q = _sc_hop(t1, pkq)
    return _tc_out(dinv, q, t1, b)


# speedup vs baseline: 1.0359x; 1.0231x over previous
import functools

import jax
import jax.numpy as jnp
from jax import lax
from jax.experimental import pallas as pl
from jax.experimental.pallas import tpu as pltpu
from jax.experimental.pallas import tpu_sc as plsc

N = 10000
E = 320000
D = 128
NC = 2
NS = 16
NW = NC * NS
NPAD = 10112
RPT = NPAD // NS
C = 128
CH = 80
EPT = C * CH
NB = NPAD // 128

_MESH = dict(core_axis_name="c", subcore_axis_name="s", num_cores=NC,
             num_subcores=NS)

_f32 = jnp.float32
_i32 = jnp.int32


@functools.partial(
    pl.kernel,
    out_type=jax.ShapeDtypeStruct((NC * NPAD,), _f32),
    mesh=plsc.VectorSubcoreMesh(**_MESH),
    scratch_types=[
        pltpu.VMEM_SHARED((NPAD,), _f32),
        pltpu.VMEM((CH, C), _i32),
        pltpu.VMEM((2, C), _i32),
        pltpu.VMEM((RPT,), _f32),
        pltpu.VMEM((C,), _f32),
        pltpu.SemaphoreType.DMA,
        pltpu.SemaphoreType.DMA,
    ],
)
def _sc_degree(pkq, degp, acc, pkv, didx, stage, ones, ssem0, ssem1):
    c = lax.axis_index("c")
    s = lax.axis_index("s")
    w = c * NS + s
    abase = pl.multiple_of(s * RPT, 8)
    obase = pl.multiple_of(c * NPAD + s * RPT, 8)
    z16 = jnp.zeros((16,), _f32)
    o16 = jnp.full((16,), 1.0, _f32)

    def fill_stage(i, _):
        stage[pl.ds(i * 16, 16)] = z16
        return 0

    lax.fori_loop(0, RPT // 16 + 1, fill_stage, 0)

    def fill_ones(i, _):
        ones[pl.ds(i * 16, 16)] = o16
        return 0

    lax.fori_loop(0, C // 16, fill_ones, 0)
    pltpu.sync_copy(stage.at[pl.ds(0, RPT)], acc.at[pl.ds(abase, RPT)])
    plsc.subcore_barrier()
    pltpu.sync_copy(pkq.at[w], pkv)

    def unpack_dst(j, b):
        for k in range(C // 16):
            didx[b, pl.ds(k * 16, 16)] = (
                lax.shift_right_logical(pkv[j, pl.ds(k * 16, 16)], 14))

    unpack_dst(0, 0)
    pltpu.async_copy(ones, acc.at[didx.at[0]], ssem0, add=True)

    def pair(i, _):
        j0 = 2 * i

        @pl.when(j0 >= 1)
        def _():
            pltpu.make_async_copy(ones, acc.at[didx.at[1]], ssem1).wait()

        unpack_dst(j0 + 1, 1)
        pltpu.async_copy(ones, acc.at[didx.at[1]], ssem1, add=True)
        pltpu.make_async_copy(ones, acc.at[didx.at[0]], ssem0).wait()

        @pl.when(j0 + 2 < CH)
        def _():
            unpack_dst(j0 + 2, 0)
            pltpu.async_copy(ones, acc.at[didx.at[0]], ssem0, add=True)

        return 0

    lax.fori_loop(0, CH // 2, pair, 0)
    pltpu.make_async_copy(ones, acc.at[didx.at[1]], ssem1).wait()
    plsc.subcore_barrier()
    pltpu.sync_copy(acc.at[pl.ds(abase, RPT)], stage.at[pl.ds(0, RPT)])
    pltpu.sync_copy(stage.at[pl.ds(0, RPT)], degp.at[pl.ds(obase, RPT)])


@functools.partial(
    pl.kernel,
    out_type=jax.ShapeDtypeStruct((NC, NPAD, D), _f32),
    mesh=plsc.VectorSubcoreMesh(**_MESH),
    scratch_types=[
        pltpu.VMEM_SHARED((NPAD, D), _f32),
        pltpu.VMEM((CH, C), _i32),
        pltpu.VMEM((2, C), _i32),
        pltpu.VMEM((2, C), _i32),
        pltpu.VMEM((2, C, D), _f32),
        pltpu.SemaphoreType.DMA,
        pltpu.SemaphoreType.DMA,
        pltpu.SemaphoreType.DMA,
        pltpu.SemaphoreType.DMA,
    ],
)
def _sc_hop(t_hbm, pkq, u_hbm, acc, pkv, sidx, didx, rbuf,
            gsem0, gsem1, ssem0, ssem1):
    c = lax.axis_index("c")
    s = lax.axis_index("s")
    w = c * NS + s
    z16 = jnp.zeros((16,), _f32)

    def unpack(j, b):
        for k in range(C // 16):
            p = pkv[j, pl.ds(k * 16, 16)]
            sidx[b, pl.ds(k * 16, 16)] = p & 16383
            didx[b, pl.ds(k * 16, 16)] = lax.shift_right_logical(p, 14)

    base = pl.multiple_of(s * RPT, 8)

    @pl.when(c == 0)
    def _():
        pltpu.sync_copy(t_hbm.at[pl.ds(base, RPT)], acc.at[pl.ds(base, RPT)])

    @pl.when(c == 1)
    def _():
        def zero_rbuf(i, _):
            rbuf[0, i // 8, pl.ds((i % 8) * 16, 16)] = z16
            return 0

        lax.fori_loop(0, C * D // 16, zero_rbuf, 0)
        for k in range(4):
            pltpu.sync_copy(rbuf.at[0], acc.at[pl.ds(base + k * 128, 128)])
        pltpu.sync_copy(rbuf.at[0, pl.ds(0, RPT - 512)],
                        acc.at[pl.ds(base + 512, RPT - 512)])

    plsc.subcore_barrier()
    pltpu.sync_copy(pkq.at[w], pkv)

    unpack(0, 0)
    pltpu.async_copy(t_hbm.at[sidx.at[0]], rbuf.at[0], gsem0)

    def pair(i, _):
        j0 = 2 * i
        pltpu.make_async_copy(t_hbm.at[sidx.at[0]], rbuf.at[0], gsem0).wait()
        pltpu.async_copy(rbuf.at[0], acc.at[didx.at[0]], ssem0, add=True)

        @pl.when(j0 >= 1)
        def _():
            pltpu.make_async_copy(rbuf.at[1], acc.at[didx.at[1]],
                                  ssem1).wait()

        unpack(j0 + 1, 1)
        pltpu.async_copy(t_hbm.at[sidx.at[1]], rbuf.at[1], gsem1)
        pltpu.make_async_copy(t_hbm.at[sidx.at[1]], rbuf.at[1], gsem1).wait()
        pltpu.async_copy(rbuf.at[1], acc.at[didx.at[1]], ssem1, add=True)
        pltpu.make_async_copy(rbuf.at[0], acc.at[didx.at[0]], ssem0).wait()

        @pl.when(j0 + 2 < CH)
        def _():
            unpack(j0 + 2, 0)
            pltpu.async_copy(t_hbm.at[sidx.at[0]], rbuf.at[0], gsem0)

        return 0

    lax.fori_loop(0, CH // 2, pair, 0)
    pltpu.make_async_copy(rbuf.at[1], acc.at[didx.at[1]], ssem1).wait()
    plsc.subcore_barrier()
    for k in range(4):
        pltpu.sync_copy(acc.at[pl.ds(base + k * 128, 128)], rbuf.at[0])
        pltpu.sync_copy(rbuf.at[0], u_hbm.at[c, pl.ds(base + k * 128, 128)])
    pltpu.sync_copy(acc.at[pl.ds(base + 512, RPT - 512)],
                    rbuf.at[0, pl.ds(0, RPT - 512)])
    pltpu.sync_copy(rbuf.at[0, pl.ds(0, RPT - 512)],
                    u_hbm.at[c, pl.ds(base + 512, RPT - 512)])


@functools.partial(
    pl.kernel,
    out_type=(
        jax.ShapeDtypeStruct((NC * NPAD, D), _f32),
        jax.ShapeDtypeStruct((NC, NPAD, D), _f32),
    ),
    mesh=plsc.VectorSubcoreMesh(**_MESH),
    scratch_types=[
        pltpu.VMEM_SHARED((NPAD, D), _f32),
        pltpu.VMEM((CH, C), _i32),
        pltpu.VMEM((2, C), _i32),
        pltpu.VMEM((2, C), _i32),
        pltpu.VMEM((2, C, D), _f32),
        pltpu.SemaphoreType.DMA,
        pltpu.SemaphoreType.DMA,
        pltpu.SemaphoreType.DMA,
        pltpu.SemaphoreType.DMA,
    ],
)
def _sc_hop2(u_hbm, dsq, pkq, t1c, q_hbm, acc, pkv, sidx, didx, rbuf,
             gsem0, gsem1, ssem0, ssem1):
    c = lax.axis_index("c")
    s = lax.axis_index("s")
    w = c * NS + s
    z16 = jnp.zeros((16,), _f32)
    base = pl.multiple_of(s * RPT, 8)
    hbase = pl.multiple_of(c * NPAD + s * RPT, 8)
    coff = c * NPAD

    def unpack(j, b):
        for k in range(C // 16):
            p = pkv[j, pl.ds(k * 16, 16)]
            sidx[b, pl.ds(k * 16, 16)] = (p & 16383) + coff
            didx[b, pl.ds(k * 16, 16)] = lax.shift_right_logical(p, 14)

    for roff, rows in ((0, 128), (128, 128), (256, 128), (384, 128),
                       (512, RPT - 512)):
        pltpu.sync_copy(u_hbm.at[0, pl.ds(base + roff, rows)],
                        rbuf.at[0, pl.ds(0, rows)])
        pltpu.sync_copy(u_hbm.at[1, pl.ds(base + roff, rows)],
                        rbuf.at[1, pl.ds(0, rows)])

        def add_rows(r, _):
            for k in range(D // 16):
                sl = pl.ds(k * 16, 16)
                rbuf[0, r, sl] = rbuf[0, r, sl] + rbuf[1, r, sl]
            return 0

        lax.fori_loop(0, rows, add_rows, 0)
        pltpu.sync_copy(dsq.at[pl.ds(base + roff, rows)],
                        rbuf.at[1, pl.ds(0, rows)])

        def mul_rows(r, _):
            for k in range(D // 16):
                sl = pl.ds(k * 16, 16)
                rbuf[0, r, sl] = rbuf[0, r, sl] * rbuf[1, r, sl]
            return 0

        lax.fori_loop(0, rows, mul_rows, 0)
        pltpu.sync_copy(rbuf.at[0, pl.ds(0, rows)],
                        t1c.at[pl.ds(hbase + roff, rows)])

        @pl.when(c == 0)
        def _():
            pltpu.sync_copy(rbuf.at[0, pl.ds(0, rows)],
                            acc.at[pl.ds(base + roff, rows)])

    @pl.when(c == 1)
    def _():
        def zero_rbuf(i, _):
            rbuf[0, i // 8, pl.ds((i % 8) * 16, 16)] = z16
            return 0

        lax.fori_loop(0, C * D // 16, zero_rbuf, 0)
        for k in range(4):
            pltpu.sync_copy(rbuf.at[0], acc.at[pl.ds(base + k * 128, 128)])
        pltpu.sync_copy(rbuf.at[0, pl.ds(0, RPT - 512)],
                        acc.at[pl.ds(base + 512, RPT - 512)])

    plsc.subcore_barrier()
    pltpu.sync_copy(pkq.at[w], pkv)

    unpack(0, 0)
    pltpu.async_copy(t1c.at[sidx.at[0]], rbuf.at[0], gsem0)

    def pair(i, _):
        j0 = 2 * i
        pltpu.make_async_copy(t1c.at[sidx.at[0]], rbuf.at[0], gsem0).wait()
        pltpu.async_copy(rbuf.at[0], acc.at[didx.at[0]], ssem0, add=True)

        @pl.when(j0 >= 1)
        def _():
            pltpu.make_async_copy(rbuf.at[1], acc.at[didx.at[1]],
                                  ssem1).wait()

        unpack(j0 + 1, 1)
        pltpu.async_copy(t1c.at[sidx.at[1]], rbuf.at[1], gsem1)
        pltpu.make_async_copy(t1c.at[sidx.at[1]], rbuf.at[1], gsem1).wait()
        pltpu.async_copy(rbuf.at[1], acc.at[didx.at[1]], ssem1, add=True)
        pltpu.make_async_copy(rbuf.at[0], acc.at[didx.at[0]], ssem0).wait()

        @pl.when(j0 + 2 < CH)
        def _():
            unpack(j0 + 2, 0)
            pltpu.async_copy(t1c.at[sidx.at[0]], rbuf.at[0], gsem0)

        return 0

    lax.fori_loop(0, CH // 2, pair, 0)
    pltpu.make_async_copy(rbuf.at[1], acc.at[didx.at[1]], ssem1).wait()
    plsc.subcore_barrier()
    for k in range(4):
        pltpu.sync_copy(acc.at[pl.ds(base + k * 128, 128)], rbuf.at[0])
        pltpu.sync_copy(rbuf.at[0], q_hbm.at[c, pl.ds(base + k * 128, 128)])
    pltpu.sync_copy(acc.at[pl.ds(base + 512, RPT - 512)],
                    rbuf.at[0, pl.ds(0, RPT - 512)])
    pltpu.sync_copy(rbuf.at[0, pl.ds(0, RPT - 512)],
                    q_hbm.at[c, pl.ds(base + 512, RPT - 512)])


def _scale1_body(x_ref, wt_ref, d0_ref, d1_ref, t0_ref, dinv_ref, dsq_ref):
    i = pl.program_id(0)
    deg = d0_ref[:] + d1_ref[:] + 1.0
    dinv = lax.rsqrt(deg)
    rows = lax.broadcasted_iota(_i32, (128, 1), 0) + i * 128
    mask = rows < N
    xw = jnp.dot(x_ref[:], wt_ref[:], preferred_element_type=_f32)
    dsq = jnp.where(mask, jnp.broadcast_to((dinv * dinv)[:, None], (128, D)),
                    0.0)
    dsq_ref[:] = dsq
    t0_ref[:] = jnp.where(mask, xw * dinv[:, None], 0.0)
    dinv_ref[:] = jnp.where(mask[:, 0], dinv, 0.0)


def _tc_scale1(x, wt, degp):
    return pl.pallas_call(
        _scale1_body,
        grid=(NB,),
        in_specs=[
            pl.BlockSpec((128, D), lambda i: (i, 0)),
            pl.BlockSpec((D, D), lambda i: (0, 0)),
            pl.BlockSpec((128,), lambda i: (i,)),
            pl.BlockSpec((128,), lambda i: (i + NB,)),
        ],
        out_specs=[
            pl.BlockSpec((128, D), lambda i: (i, 0)),
            pl.BlockSpec((128,), lambda i: (i,)),
            pl.BlockSpec((128, D), lambda i: (i, 0)),
        ],
        out_shape=[
            jax.ShapeDtypeStruct((NPAD, D), _f32),
            jax.ShapeDtypeStruct((NPAD,), _f32),
            jax.ShapeDtypeStruct((NPAD, D), _f32),
        ],
    )(x, wt, degp, degp)


def _out_body(dinv_ref, q0_ref, q1_ref, b_ref, o_ref):
    dinv = dinv_ref[:]
    o_ref[:] = dinv[:, None] * (q0_ref[0] + q1_ref[0]) + b_ref[:][None, :]


def _tc_out(dinv, q, b):
    return pl.pallas_call(
        _out_body,
        grid=(NB,),
        in_specs=[
            pl.BlockSpec((128,), lambda i: (i,)),
            pl.BlockSpec((1, 128, D), lambda i: (0, i, 0)),
            pl.BlockSpec((1, 128, D), lambda i: (1, i, 0)),
            pl.BlockSpec((D,), lambda i: (0,)),
        ],
        out_specs=pl.BlockSpec((128, D), lambda i: (i, 0)),
        out_shape=jax.ShapeDtypeStruct((N, D), _f32),
    )(dinv, q, q, b)


def kernel(x, edge_index, W, b):
    src = edge_index[0].astype(_i32)
    dst = edge_index[1].astype(_i32)
    npadrows = NPAD - N
    pad = N + (jnp.arange(NW * EPT - E, dtype=_i32) % npadrows)
    srcp = jnp.concatenate([src, pad])
    dstp = jnp.concatenate([dst, pad])
    pkq = ((dstp << 14) | srcp).reshape(NW, CH, C)
    wt = W.T

    degp = _sc_degree(pkq)
    t0, dinv, dsq = _tc_scale1(x, wt, degp)
    u = _sc_hop(t0, pkq)
    _, q = _sc_hop2(u, dsq, pkq)
    return _tc_out(dinv, q, b)
